# Initial kernel scaffold; baseline (speedup 1.0000x reference)
import jax
import jax.numpy as jnp
from jax.experimental import pallas as pl


def kernel(x, ln1_g, ln1_b, Wk, Wq, Wv, Wp, bp, ln2_g, ln2_b, Wg, W1, b1, W2, b2):
    raise NotImplementedError("write your pallas kernel here")


# speedup vs baseline: 1.4567x; 1.4567x over previous
import math

import jax
import jax.numpy as jnp
from jax.experimental import pallas as pl
from jax.experimental.pallas import tpu as pltpu

B, T, D = 128, 32, 512
H = 16
HS = D // H
E = 4
K = 2
DFF = 4 * D

NB = 8
R = NB * T
STEPS = B // NB

_GELU_C = math.sqrt(2.0 / math.pi)


def _ln(x, g, b):
    mu = jnp.mean(x, axis=-1, keepdims=True)
    xc = x - mu
    var = jnp.mean(xc * xc, axis=-1, keepdims=True)
    return xc / jnp.sqrt(var + 1e-5) * g + b


def _block_kernel(x_ref, ln1g_ref, ln1b_ref, wqkv_ref, wp_ref, bp_ref,
                  ln2g_ref, ln2b_ref, wg_ref, w1_ref, b1_ref, w2_ref, b2_ref,
                  out_ref, loss_ref):
    i = pl.program_id(0)

    x = x_ref[...].reshape(R, D)
    h = _ln(x, ln1g_ref[...], ln1b_ref[...])
    hb = h.astype(jnp.bfloat16)

    qkv = jax.lax.dot_general(hb, wqkv_ref[...], (((1,), (0,)), ((), ())),
                              preferred_element_type=jnp.float32)
    q = qkv[:, 0:D]
    k = qkv[:, D:2 * D]
    v = qkv[:, 2 * D:3 * D]

    ri = jax.lax.broadcasted_iota(jnp.int32, (R, R), 0)
    ci = jax.lax.broadcasted_iota(jnp.int32, (R, R), 1)
    same_batch = (ri // T) == (ci // T)
    causal = (ci % T) <= (ri % T)
    mask = same_batch & causal

    scale = 1.0 / math.sqrt(D)
    neg_inf = jnp.float32(-jnp.inf)

    attn_heads = []
    for hh in range(H):
        qh = q[:, hh * HS:(hh + 1) * HS].astype(jnp.bfloat16)
        kh = k[:, hh * HS:(hh + 1) * HS].astype(jnp.bfloat16)
        vh = v[:, hh * HS:(hh + 1) * HS].astype(jnp.bfloat16)
        s = jax.lax.dot_general(qh, kh, (((1,), (1,)), ((), ())),
                                preferred_element_type=jnp.float32) * scale
        s = jnp.where(mask, s, neg_inf)
        m = jnp.max(s, axis=-1, keepdims=True)
        p = jnp.exp(s - m)
        denom = jnp.sum(p, axis=-1, keepdims=True)
        p = p * (1.0 / denom)
        a = jax.lax.dot_general(p.astype(jnp.bfloat16), vh,
                                (((1,), (0,)), ((), ())),
                                preferred_element_type=jnp.float32)
        attn_heads.append(a)
    attn = jnp.concatenate(attn_heads, axis=1)

    sa = jax.lax.dot_general(attn.astype(jnp.bfloat16), wp_ref[...],
                             (((1,), (0,)), ((), ())),
                             preferred_element_type=jnp.float32) + bp_ref[...]
    x1 = x + sa

    h2 = _ln(x1, ln2g_ref[...], ln2b_ref[...])
    h2b = h2.astype(jnp.bfloat16)

    gate = jax.lax.dot_general(h2b, wg_ref[...], (((1,), (0,)), ((), ())),
                               preferred_element_type=jnp.float32)

    idx = jax.lax.broadcasted_iota(jnp.int32, (R, E), 1)
    m1 = jnp.max(gate, axis=-1, keepdims=True)
    i1 = jnp.min(jnp.where(gate == m1, idx, E), axis=-1, keepdims=True)
    g2 = jnp.where(idx == i1, neg_inf, gate)
    m2 = jnp.max(g2, axis=-1, keepdims=True)
    i2 = jnp.min(jnp.where(g2 == m2, idx, E), axis=-1, keepdims=True)
    t = jnp.exp(m2 - m1)
    w1 = 1.0 / (1.0 + t)
    w2 = t * w1
    coef = jnp.where(idx == i1, w1, 0.0) + jnp.where(idx == i2, w2, 0.0)

    part = jnp.sum(gate).reshape(1, 1)

    @pl.when(i == 0)
    def _init():
        loss_ref[...] = jnp.zeros_like(loss_ref)

    loss_ref[...] = loss_ref[...] + part

    b1 = b1_ref[...]
    b2 = b2_ref[...]
    moe = jnp.zeros((R, D), jnp.float32)
    for e in range(E):
        z = jax.lax.dot_general(h2b, w1_ref[e], (((1,), (0,)), ((), ())),
                                preferred_element_type=jnp.float32)
        z = z + b1[e:e + 1, :]
        a = 0.5 * z * (1.0 + jnp.tanh(_GELU_C * (z + 0.044715 * z * z * z)))
        o = jax.lax.dot_general(a.astype(jnp.bfloat16), w2_ref[e],
                                (((1,), (0,)), ((), ())),
                                preferred_element_type=jnp.float32)
        moe = moe + coef[:, e:e + 1] * (o + b2[e:e + 1, :])

    out_ref[...] = (x1 + moe).reshape(NB, T, D)

    @pl.when(i == STEPS - 1)
    def _fin():
        tot = loss_ref[...]
        pbar = tot / jnp.float32(B * T * E)
        loss_ref[...] = pbar * jnp.log(pbar + 0.1)


def kernel(x, ln1_g, ln1_b, Wk, Wq, Wv, Wp, bp, ln2_g, ln2_b, Wg, W1, b1, W2, b2):
    wq2 = jnp.transpose(Wq, (1, 0, 2)).reshape(D, D)
    wk2 = jnp.transpose(Wk, (1, 0, 2)).reshape(D, D)
    wv2 = jnp.transpose(Wv, (1, 0, 2)).reshape(D, D)
    wqkv = jnp.concatenate([wq2, wk2, wv2], axis=1).astype(jnp.bfloat16)
    wpb = Wp.astype(jnp.bfloat16)
    w1b = W1.astype(jnp.bfloat16)
    w2b = W2.astype(jnp.bfloat16)
    wgb = Wg.astype(jnp.bfloat16)

    full = lambda shape: pl.BlockSpec(shape, lambda i: (0,) * len(shape))

    out, loss = pl.pallas_call(
        _block_kernel,
        grid=(STEPS,),
        in_specs=[
            pl.BlockSpec((NB, T, D), lambda i: (i, 0, 0)),
            full((1, D)), full((1, D)),
            full((D, 3 * D)),
            full((D, D)), full((1, D)),
            full((1, D)), full((1, D)),
            full((D, E)),
            full((E, D, DFF)), full((E, DFF)),
            full((E, DFF, D)), full((E, D)),
        ],
        out_specs=[
            pl.BlockSpec((NB, T, D), lambda i: (i, 0, 0)),
            pl.BlockSpec((1, 1), lambda i: (0, 0)),
        ],
        out_shape=[
            jax.ShapeDtypeStruct((B, T, D), jnp.float32),
            jax.ShapeDtypeStruct((1, 1), jnp.float32),
        ],
    )(x, ln1_g.reshape(1, D), ln1_b.reshape(1, D), wqkv, wpb,
      bp.reshape(1, D), ln2_g.reshape(1, D), ln2_b.reshape(1, D), wgb,
      w1b, b1, w2b, b2)
    return out, loss[0, 0]
